# Initial kernel scaffold; baseline (speedup 1.0000x reference)
import jax
import jax.numpy as jnp
from jax.experimental import pallas as pl


def kernel(x, edge_index, W_gcn, b_gcn, W_lin, b_lin):
    raise NotImplementedError("write your pallas kernel here")


# speedup vs baseline: 13.0695x; 13.0695x over previous
import functools

import jax
import jax.numpy as jnp
from jax import lax
from jax.experimental import pallas as pl
from jax.experimental.pallas import tpu as pltpu
from jax.experimental.pallas import tpu_sc as plsc

_N = 10000
_E = 160000
_D = 256

_NC = 2
_NS = 16
_K = 128
_EPAD = ((_E + _NS * _K - 1) // (_NS * _K)) * (_NS * _K)
_CPT = _EPAD // (_NS * _K)
_EPT = _EPAD // _NS
_NROW = 10240
_RPT = _NROW // _NS
_DH = _D // 2
_BM = 400
_NB = _N // _BM

_mesh = plsc.VectorSubcoreMesh(
    core_axis_name="c", subcore_axis_name="s", num_cores=_NC, num_subcores=_NS
)


@functools.partial(
    pl.kernel,
    out_type=jax.ShapeDtypeStruct((_NROW,), jnp.float32),
    mesh=_mesh,
    scratch_types=[
        pltpu.VMEM((_EPT,), jnp.int32),
        pltpu.VMEM((_NROW,), jnp.float32),
        pltpu.VMEM((_RPT,), jnp.float32),
        pltpu.VMEM((_RPT,), jnp.float32),
        pltpu.VMEM_SHARED((_NS, _NROW), jnp.float32),
    ],
    compiler_params=pltpu.CompilerParams(needs_layout_passes=False),
)
def _deg_kernel(dst_hbm, deg_out, dvm, cnt, tbuf, psum, stage):
    c = lax.axis_index("c")
    s = lax.axis_index("s")

    @pl.when(c == 0)
    def _():
        zeros16 = jnp.zeros((16,), jnp.float32)
        ones16 = jnp.ones((16,), jnp.float32)

        def zbody(i, carry):
            cnt[pl.ds(i * 16, 16)] = zeros16
            return carry

        lax.fori_loop(0, _NROW // 16, zbody, 0)

        pltpu.sync_copy(dst_hbm.at[pl.ds(s * _EPT, _EPT)], dvm)

        def cbody(i, carry):
            idx = dvm[pl.ds(i * 16, 16)]
            plsc.addupdate_scatter(cnt, [idx], ones16)
            return carry

        lax.fori_loop(0, _EPT // 16, cbody, 0)

        pltpu.sync_copy(cnt, stage.at[s])
        plsc.subcore_barrier()

        base = s * _RPT

        def ibody(i, carry):
            psum[pl.ds(i * 16, 16)] = ones16
            return carry

        lax.fori_loop(0, _RPT // 16, ibody, 0)

        def tloop(t, carry):
            pltpu.sync_copy(stage.at[t, pl.ds(base, _RPT)], tbuf)

            def vloop(v, inner):
                psum[pl.ds(v * 16, 16)] = (
                    psum[pl.ds(v * 16, 16)] + tbuf[pl.ds(v * 16, 16)]
                )
                return inner

            lax.fori_loop(0, _RPT // 16, vloop, 0)
            return carry

        lax.fori_loop(0, _NS, tloop, 0)
        pltpu.sync_copy(psum, deg_out.at[pl.ds(base, _RPT)])


@functools.partial(
    pl.kernel,
    out_type=jax.ShapeDtypeStruct((_NC, _NROW, _DH), jnp.float32),
    mesh=_mesh,
    scratch_types=[
        pltpu.VMEM((_CPT, _K), jnp.int32),
        pltpu.VMEM((_CPT, _K), jnp.int32),
        pltpu.VMEM((_K, _DH), jnp.float32),
        pltpu.VMEM_SHARED((_NROW, _DH), jnp.float32),
        pltpu.SemaphoreType.DMA,
    ],
    compiler_params=pltpu.CompilerParams(needs_layout_passes=False),
)
def _scatter_kernel(table, srcs3, dst3, acc_out, sidx, didx, gbuf, acc, sem):
    c = lax.axis_index("c")
    s = lax.axis_index("s")

    pltpu.sync_copy(srcs3.at[c, s], sidx)
    pltpu.sync_copy(dst3.at[s], didx)

    zeros16 = jnp.zeros((16,), jnp.float32)

    def zbody(i, carry):
        gbuf[i // 8, pl.ds((i % 8) * 16, 16)] = zeros16
        return carry

    lax.fori_loop(0, _K * _DH // 16, zbody, 0)

    def zcopy(i, carry):
        pltpu.sync_copy(gbuf, acc.at[pl.ds(s * _RPT + i * _K, _K)])
        return carry

    lax.fori_loop(0, _RPT // _K, zcopy, 0)
    plsc.subcore_barrier()

    def chunk(j, carry):
        pltpu.async_copy(table.at[sidx.at[j]], gbuf, sem).wait()
        pltpu.sync_copy(gbuf, acc.at[didx.at[j]], add=True)
        return carry

    lax.fori_loop(0, _CPT, chunk, 0)
    plsc.subcore_barrier()

    pltpu.sync_copy(
        acc.at[pl.ds(s * _RPT, _RPT)], acc_out.at[c, pl.ds(s * _RPT, _RPT)]
    )


def _mm_scale_body(x_ref, w_ref, deg_ref, g_ref):
    y = jnp.dot(x_ref[...], w_ref[...], preferred_element_type=jnp.float32)
    dinv = lax.rsqrt(deg_ref[0, 0, :])
    g = y * dinv[:, None]
    g_ref[0] = g[:, :_DH]
    g_ref[1] = g[:, _DH:]


def _mm_scale(x, w_gcn, deg3):
    return pl.pallas_call(
        _mm_scale_body,
        grid=(_NB,),
        in_specs=[
            pl.BlockSpec((_BM, _D), lambda i: (i, 0)),
            pl.BlockSpec((_D, _D), lambda i: (0, 0)),
            pl.BlockSpec((1, 1, _BM), lambda i: (i, 0, 0)),
        ],
        out_specs=pl.BlockSpec((_NC, _BM, _DH), lambda i: (0, i, 0)),
        out_shape=jax.ShapeDtypeStruct((_NC, _N, _DH), jnp.float32),
    )(x, w_gcn, deg3)


def _final_body(acc_ref, g_ref, deg_ref, bg_ref, wl_ref, bl_ref, o_ref):
    accf = jnp.concatenate([acc_ref[0], acc_ref[1]], axis=1)
    gf = jnp.concatenate([g_ref[0], g_ref[1]], axis=1)
    dinv = lax.rsqrt(deg_ref[0, 0, :])
    z = (accf + gf) * dinv[:, None] + bg_ref[...]
    o = jnp.dot(z, wl_ref[...], preferred_element_type=jnp.float32) + bl_ref[...]
    o_ref[...] = jnp.maximum(o, 0.0)


def _final(acc, g2, deg3, b_gcn, w_lin, b_lin):
    return pl.pallas_call(
        _final_body,
        grid=(_NB,),
        in_specs=[
            pl.BlockSpec((_NC, _BM, _DH), lambda i: (0, i, 0)),
            pl.BlockSpec((_NC, _BM, _DH), lambda i: (0, i, 0)),
            pl.BlockSpec((1, 1, _BM), lambda i: (i, 0, 0)),
            pl.BlockSpec((1, _D), lambda i: (0, 0)),
            pl.BlockSpec((_D, _D), lambda i: (0, 0)),
            pl.BlockSpec((1, _D), lambda i: (0, 0)),
        ],
        out_specs=pl.BlockSpec((_BM, _D), lambda i: (i, 0)),
        out_shape=jax.ShapeDtypeStruct((_N, _D), jnp.float32),
    )(acc, g2, deg3, b_gcn, w_lin, b_lin)


def kernel(x, edge_index, W_gcn, b_gcn, W_lin, b_lin):
    src = edge_index[0].astype(jnp.int32)
    dst = edge_index[1].astype(jnp.int32)

    pad = _EPAD - _E
    src_p = jnp.concatenate([src, jnp.zeros((pad,), jnp.int32)])
    dst_p = jnp.concatenate([dst, jnp.full((pad,), _N, jnp.int32)])
    srcs3 = jnp.stack([src_p, src_p + _N]).reshape(_NC, _NS, _CPT, _K)
    dst3 = dst_p.reshape(_NS, _CPT, _K)

    deg = _deg_kernel(dst_p)
    deg3 = deg[:_N].reshape(_NB, 1, _BM)

    g2 = _mm_scale(x, W_gcn, deg3)
    table = g2.reshape(_NC * _N, _DH)

    acc = _scatter_kernel(table, srcs3, dst3)

    return _final(
        acc, g2, deg3, b_gcn.reshape(1, _D), W_lin, b_lin.reshape(1, _D)
    )
